# baseline (speedup 1.0000x reference)
import functools
import math

import jax
import jax.numpy as jnp
from jax import lax
from jax.experimental import pallas as pl
from jax.experimental.pallas import tpu as pltpu
from jax.experimental.pallas import tpu_sc as plsc

H = 1024
W = 2048
N = 4194304
R = H * W

_PI = math.pi

_BNB = 2048

_AT = (0.9999993329, -0.3332985605, 0.1994653599, -0.1390853351,
       0.0964200441, -0.0559098861, 0.0218612288, -0.0040540580)
_AC = (1.5707963050, -0.2145988016, 0.0889789874, -0.0501743046,
       0.0308918810, -0.0170881256, 0.0066700901, -0.0012624911)


def _coords_body(d_ref, ri_ref, w_ref):
    x = d_ref[0]
    y = d_ref[1]
    z = d_ref[2]
    a = x
    b = -z
    absa = jnp.abs(a)
    absb = jnp.abs(b)
    mx = jnp.maximum(absa, absb)
    mn = jnp.minimum(absa, absb)
    t = mn / jnp.maximum(mx, jnp.float32(1e-30))
    t2 = t * t
    p = jnp.float32(_AT[7])
    for c in (_AT[6], _AT[5], _AT[4], _AT[3], _AT[2], _AT[1], _AT[0]):
        p = p * t2 + jnp.float32(c)
    p = p * t
    r = jnp.where(absa > absb, jnp.float32(0.5 * _PI) - p, p)
    r = jnp.where(b < 0.0, jnp.float32(_PI) - r, r)
    r = jnp.where(a < 0.0, -r, r)
    u = r * jnp.float32(1.0 / _PI)
    cy = jnp.clip(y, -1.0 + 1e-6, 1.0 - 1e-6)
    q = jnp.abs(cy)
    pc = jnp.float32(_AC[7])
    for c in (_AC[6], _AC[5], _AC[4], _AC[3], _AC[2], _AC[1], _AC[0]):
        pc = pc * q + jnp.float32(c)
    ac = jnp.sqrt(jnp.maximum(1.0 - q, 0.0)) * pc
    ac = jnp.where(cy < 0.0, jnp.float32(_PI) - ac, ac)
    v = ac * jnp.float32(2.0 / _PI) - 1.0
    ix = jnp.clip(((u + 1.0) * W - 1.0) * 0.5, 0.0, W - 1.0)
    iy = jnp.clip(((v + 1.0) * H - 1.0) * 0.5, 0.0, H - 1.0)
    ix0 = jnp.floor(ix)
    iy0 = jnp.floor(iy)
    wx1 = ix - ix0
    wy1 = iy - iy0
    wx0 = 1.0 - wx1
    wy0 = 1.0 - wy1
    ri_ref[...] = iy0.astype(jnp.int32) * W + ix0.astype(jnp.int32)
    w_ref[:, 0 * _BNB:1 * _BNB] = wx0 * wy0
    w_ref[:, 1 * _BNB:2 * _BNB] = wx1 * wy0
    w_ref[:, 2 * _BNB:3 * _BNB] = wx0 * wy1
    w_ref[:, 3 * _BNB:4 * _BNB] = wx1 * wy1


def _coords(d3):
    nr = N // _BNB
    return pl.pallas_call(
        _coords_body,
        grid=(nr // 8,),
        in_specs=[pl.BlockSpec((3, 8, _BNB), lambda i: (0, i, 0))],
        out_specs=[pl.BlockSpec((8, _BNB), lambda i: (i, 0)),
                   pl.BlockSpec((8, 4 * _BNB), lambda i: (i, 0))],
        out_shape=[jax.ShapeDtypeStruct((nr, _BNB), jnp.int32),
                   jax.ShapeDtypeStruct((nr, 4 * _BNB), jnp.float32)],
    )(d3)


_NC = 2
_NS = 16
_NW = _NC * _NS
_NPW = N // _NW
_CH = 2048
_NCHUNK = _NPW // _CH
_G = _CH // 16
_RWB = H // _NS


def _sc_all_body(b2_hbm, ri_hbm, w_hbm, out_hbm, q_hbm,
                 rb0, rb1, rows0, rows1, idx0, idx1, wv0, wv1, ov0, ov1,
                 s_row, s_q0, s_q1, sidx, sg, sw, so0, so1):
    cid = lax.axis_index("c")
    sid = lax.axis_index("s")
    qtab = q_hbm.at[cid]
    lanes = lax.iota(jnp.int32, 16)

    rbs = (rb0, rb1)
    qvs = (rows0, rows1)
    sqs = (s_q0, s_q1)
    iy0 = sid * _RWB

    def start_row(iy, b):
        pltpu.async_copy(b2_hbm.at[jnp.minimum(iy, H - 1)], rbs[b], s_row)

    def wait_row():
        pltpu.make_async_copy(b2_hbm.at[0], rbs[0], s_row).wait()

    def bcompute(ba, bb):
        rowA = rbs[ba]
        rowB = rbs[bb]
        qv = qvs[ba]

        def gbody(g, carry):
            ixv = g * 16 + lanes
            i0 = ixv * 3
            i1 = jnp.minimum(ixv + 1, W - 1) * 3
            for c in range(3):
                v00 = jnp.exp(plsc.load_gather(rowA, [i0 + c]))
                v01 = jnp.exp(plsc.load_gather(rowA, [i1 + c]))
                v10 = jnp.exp(plsc.load_gather(rowB, [i0 + c]))
                v11 = jnp.exp(plsc.load_gather(rowB, [i1 + c]))
                plsc.store_scatter(qv, [ixv, jnp.full((16,), c, jnp.int32)], v00)
                plsc.store_scatter(qv, [ixv, jnp.full((16,), 3 + c, jnp.int32)], v01)
                plsc.store_scatter(qv, [ixv, jnp.full((16,), 6 + c, jnp.int32)], v10)
                plsc.store_scatter(qv, [ixv, jnp.full((16,), 9 + c, jnp.int32)], v11)
            return carry

        lax.fori_loop(0, W // 16, gbody, 0)

    def start_qout(j, b):
        r0 = pl.multiple_of((iy0 + j) * W, 8)
        pltpu.async_copy(qvs[b], qtab.at[pl.ds(r0, W), :], sqs[b])

    def wait_qout(b):
        pltpu.make_async_copy(qvs[b], qtab.at[pl.ds(0, W), :], sqs[b]).wait()

    start_row(iy0, 0)
    start_row(iy0 + 1, 1)
    wait_row()
    wait_row()
    bcompute(0, 1)
    start_row(iy0 + 2, 0)
    start_qout(0, 0)
    wait_row()
    bcompute(1, 0)
    start_row(iy0 + 3, 1)
    start_qout(1, 1)

    def jbody(j2, carry):
        for jb in (0, 1):
            j = j2 * 2 + jb
            ba = jb
            bb = 1 - jb
            wait_row()
            wait_qout(ba)
            bcompute(ba, bb)
            start_row(iy0 + j + 2, ba)
            start_qout(j, ba)
        return carry

    lax.fori_loop(1, _RWB // 2, jbody, 0)
    wait_row()
    wait_qout(0)
    wait_qout(1)

    plsc.subcore_barrier()

    wid = sid * _NC + cid
    irow0 = wid * (_NPW // 128)
    crow0 = wid * _NCHUNK
    orow0 = wid * _NPW
    idxv = (idx0, idx1)
    rowsv = (rows0, rows1)
    wvv = (wv0, wv1)
    ovv = (ov0, ov1)
    sov = (so0, so1)

    def start_idx(ci, b):
        r0 = pl.multiple_of(irow0 + ci * 16, 16)
        pltpu.async_copy(ri_hbm.at[pl.ds(r0, 16), :], idxv[b], sidx)

    def wait_idx(b):
        pltpu.make_async_copy(ri_hbm.at[pl.ds(0, 16), :], idxv[b], sidx).wait()

    def start_w(ci, b):
        pltpu.async_copy(w_hbm.at[crow0 + ci], wvv[b], sw)

    def wait_w(b):
        pltpu.make_async_copy(w_hbm.at[0], wvv[b], sw).wait()

    def fire_g(b):
        for j in range(16):
            pltpu.async_copy(qtab.at[idxv[b].at[j]],
                             rowsv[b].at[pl.ds(j * 128, 128)], sg)

    def wait_g(b):
        pltpu.make_async_copy(qtab.at[pl.ds(0, _CH)], rowsv[b], sg).wait()

    def start_out(ci, b):
        r0 = pl.multiple_of(orow0 + ci * _CH, _CH)
        for c in range(3):
            pltpu.async_copy(ovv[b].at[c], out_hbm.at[c, pl.ds(r0, _CH)], sov[b])

    def wait_out(b):
        for c in range(3):
            pltpu.make_async_copy(ovv[b].at[c], out_hbm.at[c, pl.ds(0, _CH)],
                                  sov[b]).wait()

    def compute(b):
        rv = rowsv[b]
        wv = wvv[b]
        ov = ovv[b]

        def gbody(g, carry):
            rid = g * 16 + lanes
            o16 = g * 16
            for c in range(3):
                acc = None
                for k in range(4):
                    col = jnp.full((16,), 3 * k + c, jnp.int32)
                    val = plsc.load_gather(rv, [rid, col])
                    term = val * wv[pl.ds(k * _CH + o16, 16)]
                    acc = term if acc is None else acc + term
                ov[c, pl.ds(o16, 16)] = jnp.maximum(acc, 0.0)
            return carry

        lax.fori_loop(0, _G, gbody, 0)

    start_idx(0, 0)
    start_w(0, 0)
    wait_idx(0)
    fire_g(0)

    def cbody(i2, carry):
        for b in (0, 1):
            ci = i2 * 2 + b
            cn = jnp.minimum(ci + 1, _NCHUNK - 1)
            nb = 1 - b
            start_idx(cn, nb)
            wait_g(b)
            wait_idx(nb)
            fire_g(nb)
            start_w(cn, nb)

            @pl.when(ci >= 2)
            def _():
                wait_out(b)

            wait_w(b)
            compute(b)
            start_out(ci, b)
        return carry

    lax.fori_loop(0, _NCHUNK // 2, cbody, 0)
    wait_g(0)
    wait_w(0)
    wait_out(0)
    wait_out(1)


def _sample_all(b2, ri_t, w4):
    mesh = plsc.VectorSubcoreMesh(core_axis_name="c", subcore_axis_name="s")
    fn = functools.partial(
        pl.kernel,
        out_type=[jax.ShapeDtypeStruct((3, N), jnp.float32),
                  jax.ShapeDtypeStruct((_NC, R, 16), jnp.float32)],
        mesh=mesh,
        name="sc_envmap",
        compiler_params=pltpu.CompilerParams(
            needs_layout_passes=False, use_tc_tiling_on_sc=False),
        scratch_types=[
            pltpu.VMEM((W * 3,), jnp.float32),
            pltpu.VMEM((W * 3,), jnp.float32),
            pltpu.VMEM((_CH, 16), jnp.float32),
            pltpu.VMEM((_CH, 16), jnp.float32),
            pltpu.VMEM((16, 128), jnp.int32),
            pltpu.VMEM((16, 128), jnp.int32),
            pltpu.VMEM((4 * _CH,), jnp.float32),
            pltpu.VMEM((4 * _CH,), jnp.float32),
            pltpu.VMEM((3, _CH), jnp.float32),
            pltpu.VMEM((3, _CH), jnp.float32),
            pltpu.SemaphoreType.DMA,
            pltpu.SemaphoreType.DMA,
            pltpu.SemaphoreType.DMA,
            pltpu.SemaphoreType.DMA,
            pltpu.SemaphoreType.DMA,
            pltpu.SemaphoreType.DMA,
            pltpu.SemaphoreType.DMA,
            pltpu.SemaphoreType.DMA,
        ],
    )(_sc_all_body)
    return fn(b2, ri_t, w4)


def _dtr_body(d_ref, o_ref):
    o_ref[...] = jnp.transpose(d_ref[...], (1, 0))


def _dtr(directions):
    return pl.pallas_call(
        _dtr_body,
        grid=(N // 8192,),
        in_specs=[pl.BlockSpec((8192, 3), lambda i: (i, 0))],
        out_specs=pl.BlockSpec((3, 8192), lambda i: (0, i)),
        out_shape=jax.ShapeDtypeStruct((3, N), jnp.float32),
    )(directions)


def kernel(directions, base):
    f32 = jnp.float32
    b2 = base.astype(f32).reshape(H, W * 3)
    d3 = _dtr(directions.astype(f32).reshape(N, 3)).reshape(
        3, N // _BNB, _BNB)
    ri, w4 = _coords(d3)
    ri_t = ri.reshape(N // 128, 128)

    out3t, _ = _sample_all(b2, ri_t, w4)
    return jnp.transpose(out3t, (1, 0)).reshape(directions.shape[:-1] + (3,))


# speedup vs baseline: 1.7836x; 1.7836x over previous
import functools
import math

import jax
import jax.numpy as jnp
from jax import lax
from jax.experimental import pallas as pl
from jax.experimental.pallas import tpu as pltpu
from jax.experimental.pallas import tpu_sc as plsc

H = 1024
W = 2048
N = 4194304
R = H * W

_PI = math.pi

_BNB = 2048

_AT = (0.9999993329, -0.3332985605, 0.1994653599, -0.1390853351,
       0.0964200441, -0.0559098861, 0.0218612288, -0.0040540580)
_AC = (1.5707963050, -0.2145988016, 0.0889789874, -0.0501743046,
       0.0308918810, -0.0170881256, 0.0066700901, -0.0012624911)


def _coords_body(d_ref, ri_ref, w_ref):
    x = d_ref[0]
    y = d_ref[1]
    z = d_ref[2]
    a = x
    b = -z
    absa = jnp.abs(a)
    absb = jnp.abs(b)
    mx = jnp.maximum(absa, absb)
    mn = jnp.minimum(absa, absb)
    t = mn / jnp.maximum(mx, jnp.float32(1e-30))
    t2 = t * t
    p = jnp.float32(_AT[7])
    for c in (_AT[6], _AT[5], _AT[4], _AT[3], _AT[2], _AT[1], _AT[0]):
        p = p * t2 + jnp.float32(c)
    p = p * t
    r = jnp.where(absa > absb, jnp.float32(0.5 * _PI) - p, p)
    r = jnp.where(b < 0.0, jnp.float32(_PI) - r, r)
    r = jnp.where(a < 0.0, -r, r)
    u = r * jnp.float32(1.0 / _PI)
    cy = jnp.clip(y, -1.0 + 1e-6, 1.0 - 1e-6)
    q = jnp.abs(cy)
    pc = jnp.float32(_AC[7])
    for c in (_AC[6], _AC[5], _AC[4], _AC[3], _AC[2], _AC[1], _AC[0]):
        pc = pc * q + jnp.float32(c)
    ac = jnp.sqrt(jnp.maximum(1.0 - q, 0.0)) * pc
    ac = jnp.where(cy < 0.0, jnp.float32(_PI) - ac, ac)
    v = ac * jnp.float32(2.0 / _PI) - 1.0
    ix = jnp.clip(((u + 1.0) * W - 1.0) * 0.5, 0.0, W - 1.0)
    iy = jnp.clip(((v + 1.0) * H - 1.0) * 0.5, 0.0, H - 1.0)
    ix0 = jnp.floor(ix)
    iy0 = jnp.floor(iy)
    wx1 = ix - ix0
    wy1 = iy - iy0
    wx0 = 1.0 - wx1
    wy0 = 1.0 - wy1
    ri_ref[...] = iy0.astype(jnp.int32) * W + ix0.astype(jnp.int32)
    w_ref[:, 0 * _BNB:1 * _BNB] = wx0 * wy0
    w_ref[:, 1 * _BNB:2 * _BNB] = wx1 * wy0
    w_ref[:, 2 * _BNB:3 * _BNB] = wx0 * wy1
    w_ref[:, 3 * _BNB:4 * _BNB] = wx1 * wy1


def _coords(d3):
    nr = N // _BNB
    return pl.pallas_call(
        _coords_body,
        grid=(nr // 8,),
        in_specs=[pl.BlockSpec((3, 8, _BNB), lambda i: (0, i, 0))],
        out_specs=[pl.BlockSpec((8, _BNB), lambda i: (i, 0)),
                   pl.BlockSpec((8, 4 * _BNB), lambda i: (i, 0))],
        out_shape=[jax.ShapeDtypeStruct((nr, _BNB), jnp.int32),
                   jax.ShapeDtypeStruct((nr, 4 * _BNB), jnp.float32)],
    )(d3)


_NC = 2
_NS = 16
_NW = _NC * _NS
_NPW = N // _NW
_CH = 2048
_NCHUNK = _NPW // _CH
_G = _CH // 16
_RWB = H // _NS


def _sc_all_body(b2_hbm, ri_hbm, w_hbm, out_hbm, q_hbm,
                 rb0, rb1, rows0, rows1, idx0, idx1, wv0, wv1, ov0, ov1,
                 s_row, s_q0, s_q1, sidx, sg, sw, so0, so1):
    cid = lax.axis_index("c")
    sid = lax.axis_index("s")
    qtab = q_hbm.at[cid]
    lanes = lax.iota(jnp.int32, 16)

    rbs = (rb0, rb1)
    qvs = (rows0, rows1)
    sqs = (s_q0, s_q1)
    iy0 = sid * _RWB

    def start_row(iy, b):
        pltpu.async_copy(b2_hbm.at[jnp.minimum(iy, H - 1)], rbs[b], s_row)

    def wait_row():
        pltpu.make_async_copy(b2_hbm.at[0], rbs[0], s_row).wait()

    def bcompute(ba, bb):
        rowA = rbs[ba]
        rowB = rbs[bb]
        qv = qvs[ba]

        def gbody(g, carry):
            ixv = g * 16 + lanes
            i0 = ixv * 3
            i1 = jnp.minimum(ixv + 1, W - 1) * 3
            for c in range(3):
                v00 = jnp.exp(plsc.load_gather(rowA, [i0 + c]))
                v01 = jnp.exp(plsc.load_gather(rowA, [i1 + c]))
                v10 = jnp.exp(plsc.load_gather(rowB, [i0 + c]))
                v11 = jnp.exp(plsc.load_gather(rowB, [i1 + c]))
                plsc.store_scatter(qv, [ixv, jnp.full((16,), c, jnp.int32)], v00)
                plsc.store_scatter(qv, [ixv, jnp.full((16,), 3 + c, jnp.int32)], v01)
                plsc.store_scatter(qv, [ixv, jnp.full((16,), 6 + c, jnp.int32)], v10)
                plsc.store_scatter(qv, [ixv, jnp.full((16,), 9 + c, jnp.int32)], v11)
            return carry

        lax.fori_loop(0, W // 16, gbody, 0)

    def start_qout(j, b):
        r0 = pl.multiple_of((iy0 + j) * W, 8)
        pltpu.async_copy(qvs[b], qtab.at[pl.ds(r0, W), :], sqs[b])

    def wait_qout(b):
        pltpu.make_async_copy(qvs[b], qtab.at[pl.ds(0, W), :], sqs[b]).wait()

    start_row(iy0, 0)
    start_row(iy0 + 1, 1)
    wait_row()
    wait_row()
    bcompute(0, 1)
    start_row(iy0 + 2, 0)
    start_qout(0, 0)
    wait_row()
    bcompute(1, 0)
    start_row(iy0 + 3, 1)
    start_qout(1, 1)

    def jbody(j2, carry):
        for jb in (0, 1):
            j = j2 * 2 + jb
            ba = jb
            bb = 1 - jb
            wait_row()
            wait_qout(ba)
            bcompute(ba, bb)
            start_row(iy0 + j + 2, ba)
            start_qout(j, ba)
        return carry

    lax.fori_loop(1, _RWB // 2, jbody, 0)
    wait_row()
    wait_qout(0)
    wait_qout(1)

    plsc.subcore_barrier()

    wid = sid * _NC + cid
    irow0 = wid * (_NPW // 128)
    crow0 = wid * _NCHUNK
    orow0 = wid * _NPW
    idxv = (idx0, idx1)
    rowsv = (rows0, rows1)
    wvv = (wv0, wv1)
    ovv = (ov0, ov1)
    sov = (so0, so1)

    def start_idx(ci, b):
        r0 = pl.multiple_of(irow0 + ci * 16, 16)
        pltpu.async_copy(ri_hbm.at[pl.ds(r0, 16), :], idxv[b], sidx)

    def wait_idx(b):
        pltpu.make_async_copy(ri_hbm.at[pl.ds(0, 16), :], idxv[b], sidx).wait()

    def start_w(ci, b):
        pltpu.async_copy(w_hbm.at[crow0 + ci], wvv[b], sw)

    def wait_w(b):
        pltpu.make_async_copy(w_hbm.at[0], wvv[b], sw).wait()

    def fire_g(b):
        for j in range(16):
            pltpu.async_copy(qtab.at[idxv[b].at[j]],
                             rowsv[b].at[pl.ds(j * 128, 128)], sg)

    def wait_g(b):
        pltpu.make_async_copy(qtab.at[pl.ds(0, _CH)], rowsv[b], sg).wait()

    def start_out(ci, b):
        r0 = pl.multiple_of(orow0 + ci * _CH, _CH)
        for c in range(3):
            pltpu.async_copy(ovv[b].at[c], out_hbm.at[c, pl.ds(r0, _CH)], sov[b])

    def wait_out(b):
        for c in range(3):
            pltpu.make_async_copy(ovv[b].at[c], out_hbm.at[c, pl.ds(0, _CH)],
                                  sov[b]).wait()

    def compute(b):
        rv = rowsv[b]
        wv = wvv[b]
        ov = ovv[b]

        def gbody(g, carry):
            rid = g * 16 + lanes
            o16 = g * 16
            for c in range(3):
                acc = None
                for k in range(4):
                    col = jnp.full((16,), 3 * k + c, jnp.int32)
                    val = plsc.load_gather(rv, [rid, col])
                    term = val * wv[pl.ds(k * _CH + o16, 16)]
                    acc = term if acc is None else acc + term
                ov[c, pl.ds(o16, 16)] = jnp.maximum(acc, 0.0)
            return carry

        lax.fori_loop(0, _G, gbody, 0)

    start_idx(0, 0)
    start_w(0, 0)
    wait_idx(0)
    fire_g(0)

    def cbody(i2, carry):
        for b in (0, 1):
            ci = i2 * 2 + b
            cn = jnp.minimum(ci + 1, _NCHUNK - 1)
            nb = 1 - b
            start_idx(cn, nb)
            wait_g(b)
            wait_idx(nb)
            fire_g(nb)
            start_w(cn, nb)

            @pl.when(ci >= 2)
            def _():
                wait_out(b)

            wait_w(b)
            compute(b)
            start_out(ci, b)
        return carry

    lax.fori_loop(0, _NCHUNK // 2, cbody, 0)
    wait_g(0)
    wait_w(0)
    wait_out(0)
    wait_out(1)


def _sample_all(b2, ri_t, w4):
    mesh = plsc.VectorSubcoreMesh(core_axis_name="c", subcore_axis_name="s")
    fn = functools.partial(
        pl.kernel,
        out_type=[jax.ShapeDtypeStruct((3, N), jnp.float32),
                  jax.ShapeDtypeStruct((_NC, R, 16), jnp.float32)],
        mesh=mesh,
        name="sc_envmap",
        compiler_params=pltpu.CompilerParams(
            needs_layout_passes=False, use_tc_tiling_on_sc=False),
        scratch_types=[
            pltpu.VMEM((W * 3,), jnp.float32),
            pltpu.VMEM((W * 3,), jnp.float32),
            pltpu.VMEM((_CH, 16), jnp.float32),
            pltpu.VMEM((_CH, 16), jnp.float32),
            pltpu.VMEM((16, 128), jnp.int32),
            pltpu.VMEM((16, 128), jnp.int32),
            pltpu.VMEM((4 * _CH,), jnp.float32),
            pltpu.VMEM((4 * _CH,), jnp.float32),
            pltpu.VMEM((3, _CH), jnp.float32),
            pltpu.VMEM((3, _CH), jnp.float32),
            pltpu.SemaphoreType.DMA,
            pltpu.SemaphoreType.DMA,
            pltpu.SemaphoreType.DMA,
            pltpu.SemaphoreType.DMA,
            pltpu.SemaphoreType.DMA,
            pltpu.SemaphoreType.DMA,
            pltpu.SemaphoreType.DMA,
            pltpu.SemaphoreType.DMA,
        ],
    )(_sc_all_body)
    return fn(b2, ri_t, w4)


def kernel(directions, base):
    f32 = jnp.float32
    b2 = base.astype(f32).reshape(H, W * 3)
    d3 = jnp.transpose(directions.astype(f32).reshape(N, 3),
                       (1, 0)).reshape(3, N // _BNB, _BNB)
    ri, w4 = _coords(d3)
    ri_t = ri.reshape(N // 128, 128)

    out3t, _ = _sample_all(b2, ri_t, w4)
    return jnp.transpose(out3t, (1, 0)).reshape(directions.shape[:-1] + (3,))
